# baseline (speedup 1.0000x reference)
import functools

import jax
import jax.numpy as jnp
from jax import lax
from jax.experimental import pallas as pl
from jax.experimental.pallas import tpu as pltpu
from jax.experimental.pallas import tpu_sc as plsc

VOCAB = 1_000_000
D = 64
BATCH = 4096
SEQ = 200
TOTAL = BATCH * SEQ

NC = 2
NS = 16
NW = NC * NS
PER_W = TOTAL // NW

G = 128
CHUNK_G = 4
CHUNK = G * CHUNK_G
GROUPS_PER_W = PER_W // G
NCHUNK = GROUPS_PER_W // CHUNK_G

_mesh = plsc.VectorSubcoreMesh(
    core_axis_name="c", subcore_axis_name="s", num_cores=NC, num_subcores=NS
)


@functools.partial(
    pl.kernel,
    out_type=jax.ShapeDtypeStruct((TOTAL, D), jnp.float32),
    mesh=_mesh,
    scratch_types=[
        pltpu.VMEM((CHUNK_G, G), jnp.int32),
        pltpu.VMEM((CHUNK, D), jnp.float32),
        pltpu.SemaphoreType.DMA,
    ],
    compiler_params=pltpu.CompilerParams(use_tc_tiling_on_sc=False),
)
def _sc_gather(table_hbm, idx_hbm, out_hbm, idx_v, rows_v, sem):
    wid = lax.axis_index("s") * NC + lax.axis_index("c")
    base_g = wid * GROUPS_PER_W

    def chunk_body(i, carry):
        g0 = base_g + i * CHUNK_G
        pltpu.sync_copy(idx_hbm.at[pl.ds(g0, CHUNK_G)], idx_v)
        copies = [
            pltpu.async_copy(
                table_hbm.at[idx_v.at[j]], rows_v.at[pl.ds(j * G, G)], sem
            )
            for j in range(CHUNK_G)
        ]
        for c in copies:
            c.wait()
        pltpu.sync_copy(rows_v, out_hbm.at[pl.ds(g0 * G, CHUNK)])
        return carry

    lax.fori_loop(0, NCHUNK, chunk_body, 0)


def kernel(x, table):
    idx2d = x.reshape(TOTAL // G, G)
    out = _sc_gather(table, idx2d)
    return out.reshape(BATCH, SEQ, D)


# speedup vs baseline: 1.0433x; 1.0433x over previous
import functools

import jax
import jax.numpy as jnp
from jax import lax
from jax.experimental import pallas as pl
from jax.experimental.pallas import tpu as pltpu
from jax.experimental.pallas import tpu_sc as plsc

VOCAB = 1_000_000
D = 64
BATCH = 4096
SEQ = 200
TOTAL = BATCH * SEQ

NC = 2
NS = 16
NW = NC * NS
PER_W = TOTAL // NW

G = 128
CHUNK_G = 4
CHUNK = G * CHUNK_G
GROUPS_PER_W = PER_W // G
NCHUNK = GROUPS_PER_W // CHUNK_G
NBUF = 3

assert NCHUNK >= 4

_mesh = plsc.VectorSubcoreMesh(
    core_axis_name="c", subcore_axis_name="s", num_cores=NC, num_subcores=NS
)


@functools.partial(
    pl.kernel,
    out_type=jax.ShapeDtypeStruct((TOTAL, D), jnp.float32),
    mesh=_mesh,
    scratch_types=[
        pltpu.VMEM((GROUPS_PER_W, G), jnp.int32),
        pltpu.VMEM((NBUF, CHUNK, D), jnp.float32),
        pltpu.SemaphoreType.DMA,
        pltpu.SemaphoreType.DMA,
        pltpu.SemaphoreType.DMA,
        pltpu.SemaphoreType.DMA,
        pltpu.SemaphoreType.DMA,
        pltpu.SemaphoreType.DMA,
    ],
    compiler_params=pltpu.CompilerParams(use_tc_tiling_on_sc=False),
)
def _sc_gather(table_hbm, idx_hbm, out_hbm, idx_v, rows_v, g0, g1, g2, s0, s1, s2):
    gat_sems = (g0, g1, g2)
    st_sems = (s0, s1, s2)
    wid = lax.axis_index("s") * NC + lax.axis_index("c")
    base_g = wid * GROUPS_PER_W

    pltpu.sync_copy(idx_hbm.at[pl.ds(base_g, GROUPS_PER_W)], idx_v)

    def issue_gathers(i, b):
        for j in range(CHUNK_G):
            pltpu.async_copy(
                table_hbm.at[idx_v.at[i * CHUNK_G + j]],
                rows_v.at[b, pl.ds(j * G, G)],
                gat_sems[b],
            )

    def wait_gathers(b):
        for j in range(CHUNK_G):
            pltpu.make_async_copy(
                table_hbm.at[pl.ds(0, G)],
                rows_v.at[b, pl.ds(j * G, G)],
                gat_sems[b],
            ).wait()

    def issue_store(i, b):
        pltpu.async_copy(
            rows_v.at[b],
            out_hbm.at[pl.ds((base_g + i * CHUNK_G) * G, CHUNK)],
            st_sems[b],
        )

    def wait_store(b):
        pltpu.make_async_copy(
            rows_v.at[b], out_hbm.at[pl.ds(0, CHUNK)], st_sems[b]
        ).wait()

    issue_gathers(0, 0)
    issue_gathers(1, 1)

    wait_gathers(0)
    issue_store(0, 0)
    issue_gathers(2, 2)

    def slot(i, bufs):
        b, bp = bufs
        wait_gathers(b)
        issue_store(i, b)
        wait_store(bp)
        issue_gathers(i + 2, bp)

    def body3(k, carry):
        i = 1 + 3 * k
        slot(i, (1, 0))
        slot(i + 1, (2, 1))
        slot(i + 2, (0, 2))
        return carry

    n3 = (NCHUNK - 3) // 3
    lax.fori_loop(0, n3, body3, 0)
    rem = 1 + 3 * n3
    for i in range(rem, NCHUNK - 2):
        slot(i, (i % 3, (i - 1) % 3))

    for i in range(NCHUNK - 2, NCHUNK):
        b = i % 3
        wait_gathers(b)
        issue_store(i, b)
        wait_store((i - 1) % 3)
    wait_store((NCHUNK - 1) % 3)


def kernel(x, table):
    idx2d = x.reshape(TOTAL // G, G)
    out = _sc_gather(table, idx2d)
    return out.reshape(BATCH, SEQ, D)
